# baseline (speedup 1.0000x reference)
import functools

import jax
import jax.numpy as jnp
from jax import lax
from jax.experimental import pallas as pl
from jax.experimental.pallas import tpu as pltpu
from jax.experimental.pallas import tpu_sc as plsc

NUM_ENT = 100000
NUM_REL = 1000
D = 64
BATCH = 16384
CHUNK = 4096
NCHUNK = BATCH // CHUNK
NITER = CHUNK // 16
ROWS_PER_W = 2


@functools.cache
def _make_gather2():
    info = plsc.get_sparse_core_info()
    nc = info.num_cores
    mesh = plsc.VectorSubcoreMesh(core_axis_name="c", subcore_axis_name="s")

    @functools.partial(
        pl.kernel,
        mesh=mesh,
        compiler_params=pltpu.CompilerParams(needs_layout_passes=False,
                                             vmem_limit_bytes=1 << 20),
        out_type=[
            jax.ShapeDtypeStruct((D, BATCH), jnp.float32),
            jax.ShapeDtypeStruct((D, BATCH), jnp.float32),
            jax.ShapeDtypeStruct((D, NUM_ENT), jnp.float32),
            jax.ShapeDtypeStruct((D, NUM_REL), jnp.float32),
        ],
        scratch_types=[
            pltpu.VMEM((NUM_ENT,), jnp.float32),
            pltpu.VMEM((ROWS_PER_W, NUM_REL), jnp.float32),
            pltpu.VMEM((2, CHUNK), jnp.int32),
            pltpu.VMEM((2, ROWS_PER_W, CHUNK), jnp.float32),
            pltpu.SemaphoreType.DMA,
            pltpu.SemaphoreType.DMA,
            pltpu.SemaphoreType.DMA,
            pltpu.SemaphoreType.DMA,
        ],
    )
    def gatherT(entT, relT, sub_hbm, rel_hbm, outS, outR, outE, outL,
                rowv, relv, idxv, outv,
                sem_row, sem_idx, sem_out, sem_wb):
        wid = lax.axis_index("s") * nc + lax.axis_index("c")
        d0 = wid * ROWS_PER_W
        out_pending = []

        def claim(slot):
            if any(s == slot for s, _ in out_pending):
                while out_pending:
                    out_pending.pop(0)[1].wait()

        def rel_group(first):
            ch = pltpu.async_copy(rel_hbm.at[pl.ds(first * CHUNK, CHUNK)],
                                  idxv.at[first & 1], sem_idx)
            r0 = jnp.full((16,), 0, jnp.int32)
            r1 = jnp.full((16,), 1, jnp.int32)
            for c in (first, first + 1):
                b = c & 1
                ch.wait()
                if c == first:
                    ch = pltpu.async_copy(
                        rel_hbm.at[pl.ds((c + 1) * CHUNK, CHUNK)],
                        idxv.at[1 - b], sem_idx)
                claim((b, 0))
                claim((b, 1))

                @plsc.parallel_loop(0, NITER, 1, unroll=8)
                def _(j):
                    iv = idxv[b, pl.ds(j * 16, 16)]
                    outv[b, 0, pl.ds(j * 16, 16)] = plsc.load_gather(
                        relv, [r0, iv])
                    outv[b, 1, pl.ds(j * 16, 16)] = plsc.load_gather(
                        relv, [r1, iv])

                for ri in range(ROWS_PER_W):
                    out_pending.append(((b, ri), pltpu.async_copy(
                        outv.at[b, ri],
                        outR.at[d0 + ri, pl.ds(c * CHUNK, CHUNK)], sem_out)))

        def ent_chunks(ri, row_dma):
            ci = pltpu.async_copy(sub_hbm.at[pl.ds(0, CHUNK)], idxv.at[0],
                                  sem_idx)
            row_dma.wait()
            wb = pltpu.async_copy(rowv, outE.at[d0 + ri], sem_wb)
            for c in range(NCHUNK):
                b = c & 1
                ci.wait()
                if c + 1 < NCHUNK:
                    ci = pltpu.async_copy(
                        sub_hbm.at[pl.ds((c + 1) * CHUNK, CHUNK)],
                        idxv.at[1 - b], sem_idx)
                claim((b, ri))

                @plsc.parallel_loop(0, NITER, 1, unroll=16)
                def _(j):
                    iv = idxv[b, pl.ds(j * 16, 16)]
                    outv[b, ri, pl.ds(j * 16, 16)] = plsc.load_gather(rowv, [iv])

                out_pending.append(((b, ri), pltpu.async_copy(
                    outv.at[b, ri], outS.at[d0 + ri, pl.ds(c * CHUNK, CHUNK)],
                    sem_out)))
            return wb

        ce = pltpu.async_copy(entT.at[d0], rowv, sem_row)
        for ri in range(ROWS_PER_W):
            pltpu.sync_copy(relT.at[d0 + ri], relv.at[ri])
        wbl = [pltpu.async_copy(relv.at[ri], outL.at[d0 + ri], sem_wb)
               for ri in range(ROWS_PER_W)]
        rel_group(0)
        wb = ent_chunks(0, ce)
        wb.wait()
        ce = pltpu.async_copy(entT.at[d0 + 1], rowv, sem_row)
        rel_group(2)
        wb = ent_chunks(1, ce)
        wb.wait()
        for h in wbl:
            h.wait()
        while out_pending:
            out_pending.pop(0)[1].wait()

    return gatherT


def kernel(init_embed1, init_rel1, We, Wr, sub, rel, downsample):
    outS, outR, outE, outL = _make_gather2()(
        init_embed1.T, init_rel1.T, sub.astype(jnp.int32), rel.astype(jnp.int32))
    return (outS.T, outR.T, outE.T, outL.T)


# speedup vs baseline: 1.0149x; 1.0149x over previous
import functools

import jax
import jax.numpy as jnp
from jax import lax
from jax.experimental import pallas as pl
from jax.experimental.pallas import tpu as pltpu
from jax.experimental.pallas import tpu_sc as plsc

NUM_ENT = 100000
NUM_REL = 1000
D = 64
BATCH = 16384
CHUNK = 4096
NCHUNK = BATCH // CHUNK
NITER = CHUNK // 16
ROWS_PER_W = 2


@functools.cache
def _make_gather2():
    info = plsc.get_sparse_core_info()
    nc = info.num_cores
    mesh = plsc.VectorSubcoreMesh(core_axis_name="c", subcore_axis_name="s")

    @functools.partial(
        pl.kernel,
        mesh=mesh,
        compiler_params=pltpu.CompilerParams(needs_layout_passes=False,
                                             vmem_limit_bytes=1 << 20),
        out_type=[
            jax.ShapeDtypeStruct((D, BATCH), jnp.float32),
            jax.ShapeDtypeStruct((D, BATCH), jnp.float32),
            jax.ShapeDtypeStruct((D, NUM_ENT), jnp.float32),
            jax.ShapeDtypeStruct((D, NUM_REL), jnp.float32),
        ],
        scratch_types=[
            pltpu.VMEM((NUM_ENT,), jnp.float32),
            pltpu.VMEM((ROWS_PER_W, NUM_REL), jnp.float32),
            pltpu.VMEM((2, CHUNK), jnp.int32),
            pltpu.VMEM((2, ROWS_PER_W, CHUNK), jnp.float32),
            pltpu.SemaphoreType.DMA,
            pltpu.SemaphoreType.DMA,
            pltpu.SemaphoreType.DMA,
            pltpu.SemaphoreType.DMA,
        ],
    )
    def gatherT(entT, relT, sub_hbm, rel_hbm, outS, outR, outE, outL,
                rowv, relv, idxv, outv,
                sem_row, sem_idx, sem_out, sem_wb):
        wid = lax.axis_index("s") * nc + lax.axis_index("c")
        d0 = wid * ROWS_PER_W
        out_pending = []

        def claim(slot):
            if any(s == slot for s, _ in out_pending):
                while out_pending:
                    out_pending.pop(0)[1].wait()

        def rel_group(first):
            ch = pltpu.async_copy(rel_hbm.at[pl.ds(first * CHUNK, CHUNK)],
                                  idxv.at[first & 1], sem_idx)
            r0 = jnp.full((16,), 0, jnp.int32)
            r1 = jnp.full((16,), 1, jnp.int32)
            for c in (first, first + 1):
                b = c & 1
                ch.wait()
                if c == first:
                    ch = pltpu.async_copy(
                        rel_hbm.at[pl.ds((c + 1) * CHUNK, CHUNK)],
                        idxv.at[1 - b], sem_idx)
                claim((b, 0))
                claim((b, 1))

                @plsc.parallel_loop(0, NITER, 1, unroll=8)
                def _(j):
                    iv = idxv[b, pl.ds(j * 16, 16)]
                    outv[b, 0, pl.ds(j * 16, 16)] = plsc.load_gather(
                        relv, [r0, iv])
                    outv[b, 1, pl.ds(j * 16, 16)] = plsc.load_gather(
                        relv, [r1, iv])

                for ri in range(ROWS_PER_W):
                    out_pending.append(((b, ri), pltpu.async_copy(
                        outv.at[b, ri],
                        outR.at[d0 + ri, pl.ds(c * CHUNK, CHUNK)], sem_out)))

        def ent_chunks(ri, row_dma):
            ci = pltpu.async_copy(sub_hbm.at[pl.ds(0, CHUNK)], idxv.at[0],
                                  sem_idx)
            row_dma.wait()
            wb = pltpu.async_copy(rowv, outE.at[d0 + ri], sem_wb)
            for c in range(NCHUNK):
                b = c & 1
                ci.wait()
                if c + 1 < NCHUNK:
                    ci = pltpu.async_copy(
                        sub_hbm.at[pl.ds((c + 1) * CHUNK, CHUNK)],
                        idxv.at[1 - b], sem_idx)
                claim((b, ri))

                @plsc.parallel_loop(0, NITER, 1, unroll=8)
                def _(j):
                    iv = idxv[b, pl.ds(j * 16, 16)]
                    outv[b, ri, pl.ds(j * 16, 16)] = plsc.load_gather(rowv, [iv])

                out_pending.append(((b, ri), pltpu.async_copy(
                    outv.at[b, ri], outS.at[d0 + ri, pl.ds(c * CHUNK, CHUNK)],
                    sem_out)))
            return wb

        ce = pltpu.async_copy(entT.at[d0], rowv, sem_row)
        for ri in range(ROWS_PER_W):
            pltpu.sync_copy(relT.at[d0 + ri], relv.at[ri])
        wbl = [pltpu.async_copy(relv.at[ri], outL.at[d0 + ri], sem_wb)
               for ri in range(ROWS_PER_W)]
        rel_group(0)
        wb = ent_chunks(0, ce)
        wb.wait()
        ce = pltpu.async_copy(entT.at[d0 + 1], rowv, sem_row)
        rel_group(2)
        wb = ent_chunks(1, ce)
        wb.wait()
        for h in wbl:
            h.wait()
        while out_pending:
            out_pending.pop(0)[1].wait()

    return gatherT


def kernel(init_embed1, init_rel1, We, Wr, sub, rel, downsample):
    outS, outR, outE, outL = _make_gather2()(
        init_embed1.T, init_rel1.T, sub.astype(jnp.int32), rel.astype(jnp.int32))
    return (outS.T, outR.T, outE.T, outL.T)
